# baseline (speedup 1.0000x reference)
import jax
import jax.numpy as jnp
from jax import lax
from jax.experimental import pallas as pl
from jax.experimental.pallas import tpu as pltpu
from jax.experimental.pallas import tpu_sc as plsc

N = 10000
CH = 128
H = 64
NS = 16
K = 128
NBUF = 4
LOOK = 4
SUP = 8
MB = 3
WB_CHUNK = 80
N_WB_CHUNKS = N // WB_CHUNK
ILV = plsc.PackFormat.INTERLEAVED


def _pack_edges(src, dst, val, total, src_mod):
    pad = total - src.shape[0]
    pad_idx = jnp.arange(pad, dtype=jnp.int32)
    src = jnp.concatenate([src.astype(jnp.int32), pad_idx % src_mod])
    dst = jnp.concatenate([dst.astype(jnp.int32), pad_idx % N])
    val = jnp.concatenate([val, jnp.zeros((pad,), val.dtype)])
    vbits = jax.lax.bitcast_convert_type(val.astype(jnp.float32), jnp.int32)
    nsc = total // (NS * K * SUP)
    packed = jnp.stack([x.reshape(NS * nsc, SUP, K) for x in (src, dst, vbits)],
                       axis=1)
    return packed


def _body(m1, m2, m3, w2, diag,
          out_ref, h1, h2,
          w_sp, acc,
          gbuf0, gbuf1, gbuf2, gbuf3,
          rows0, rows1, rows2, rows3,
          cbuf, dbuf, zbuf, hbuf,
          gsem0, gsem1, gsem2, gsem3,
          ssem0, ssem1, ssem2, ssem3, msem):
    c = lax.axis_index("c")
    s = lax.axis_index("s")
    gbufs = (gbuf0, gbuf1, gbuf2, gbuf3)
    rows_list = (rows0, rows1, rows2, rows3)
    gsems = (gsem0, gsem1, gsem2, gsem3)
    ssems = (ssem0, ssem1, ssem2, ssem3)
    n_rounds = (N_WB_CHUNKS + NS - 1) // NS

    @pl.when(s == 0)
    def _():
        pltpu.sync_copy(w2.at[c], gbuf0)
        pltpu.sync_copy(gbuf0, w_sp)

    def zb(r, _):
        z = jnp.zeros((16,), jnp.float32)
        for q in range(H // 16):
            zbuf[r, pl.ds(q * 16, 16)] = z
        return _
    lax.fori_loop(0, WB_CHUNK, zb, None)

    for j in range(n_rounds):
        ci = j * NS + s

        @pl.when(ci < N_WB_CHUNKS)
        def _():
            pltpu.sync_copy(zbuf, acc.at[pl.ds(ci * WB_CHUNK, WB_CHUNK)])

    plsc.subcore_barrier()

    def _wait_gather(b):
        pltpu.make_async_copy(w2.at[c], gbufs[b], gsems[b]).wait()

    def _wait_scatter(b):
        pltpu.make_async_copy(out_ref.at[c, pl.ds(0, K)], rows_list[b],
                              ssems[b]).wait()

    def _stage(nsc, meta, table_ref):
        nc = nsc * SUP
        base = s * nsc
        pltpu.sync_copy(meta.at[base], cbuf.at[0])
        if nsc > 1:
            pltpu.async_copy(meta.at[base + 1], cbuf.at[1], msem)

        for b in range(LOOK):
            pltpu.async_copy(table_ref.at[cbuf.at[0, 0, b]], gbufs[b],
                             gsems[b])

        def super_body(sb, _):
            b2 = lax.rem(sb, MB)
            b2n = lax.rem(sb + 1, MB)
            for j in range(SUP):
                k = sb * SUP + j
                b = j % NBUF
                _wait_gather(b)
                @pl.when(k >= LOOK)
                def _():
                    _wait_scatter(b)

                if j == 3:
                    @pl.when(sb + 1 < nsc)
                    def _():
                        pltpu.make_async_copy(meta.at[base], cbuf.at[0],
                                              msem).wait()

                    @pl.when(sb + 2 < nsc)
                    def _():
                        pltpu.async_copy(meta.at[base + sb + 2],
                                         cbuf.at[lax.rem(sb + 2, MB)], msem)
                def sc(jj, _):
                    v16 = plsc.bitcast(cbuf[b2, 2, j, pl.ds(jj * 16, 16)],
                                       jnp.float32)
                    for i in range(16):
                        sv = v16[jnp.full((16,), i, jnp.int32)]
                        r = jj * 16 + i
                        for q in range(H // 32):
                            x32 = gbufs[b][r, pl.ds(q * 32, 32)]
                            lo, hi = plsc.unpack(x32, format=ILV)
                            rows_list[b][r, pl.ds(q * 32, 16)] = lo * sv
                            rows_list[b][r, pl.ds(q * 32 + 16, 16)] = hi * sv
                    return _
                lax.fori_loop(0, K // 16, sc, None)
                if j < SUP - LOOK:
                    pltpu.async_copy(table_ref.at[cbuf.at[b2, 0, j + LOOK]],
                                     gbufs[b], gsems[b])
                else:
                    @pl.when(sb + 1 < nsc)
                    def _():
                        pltpu.async_copy(
                            table_ref.at[cbuf.at[b2n, 0, j + LOOK - SUP]],
                            gbufs[b], gsems[b])
                pltpu.async_copy(rows_list[b], acc.at[cbuf.at[b2, 1, j]],
                                 ssems[b], add=True)
            return _
        lax.fori_loop(0, nsc, super_body, None)

        for k in range(nc - LOOK, nc):
            _wait_scatter(k % NBUF)

    def _pack80():
        def pk(r, _):
            for q in range(H // 32):
                lo = rows0[r, pl.ds(q * 32, 16)]
                hi = rows0[r, pl.ds(q * 32 + 16, 16)]
                hbuf[r, pl.ds(q * 32, 32)] = plsc.pack(lo, hi, format=ILV)
            return _
        lax.fori_loop(0, WB_CHUNK, pk, None)

    nsc1 = m1.shape[0] // NS
    nsc2 = m2.shape[0] // NS

    _stage(nsc1, m1, w_sp)
    plsc.subcore_barrier()
    for j in range(n_rounds):
        ci = j * NS + s

        @pl.when(ci < N_WB_CHUNKS)
        def _():
            r0 = ci * WB_CHUNK
            pltpu.sync_copy(acc.at[pl.ds(r0, WB_CHUNK)],
                            rows0.at[pl.ds(0, WB_CHUNK)])
            _pack80()
            pltpu.sync_copy(hbuf, h1.at[c, pl.ds(r0, WB_CHUNK)])
            pltpu.sync_copy(zbuf, acc.at[pl.ds(r0, WB_CHUNK)])

    plsc.subcore_barrier()
    _stage(nsc2, m2, h1.at[c])
    plsc.subcore_barrier()
    for j in range(n_rounds):
        ci = j * NS + s

        @pl.when(ci < N_WB_CHUNKS)
        def _():
            r0 = ci * WB_CHUNK
            pltpu.sync_copy(acc.at[pl.ds(r0, WB_CHUNK)],
                            rows0.at[pl.ds(0, WB_CHUNK)])
            pltpu.sync_copy(diag.at[pl.ds(r0, WB_CHUNK)], dbuf)

            def dsc(jj, _):
                d16 = dbuf[pl.ds(jj * 16, 16)]
                for i in range(16):
                    sv = d16[jnp.full((16,), i, jnp.int32)]
                    r = jj * 16 + i
                    for q in range(H // 16):
                        sl = pl.ds(q * 16, 16)
                        rows0[r, sl] = rows0[r, sl] * sv
                return _
            lax.fori_loop(0, WB_CHUNK // 16, dsc, None)

            _pack80()
            pltpu.sync_copy(hbuf, h2.at[c, pl.ds(r0, WB_CHUNK)])
            pltpu.sync_copy(zbuf, acc.at[pl.ds(r0, WB_CHUNK)])

    plsc.subcore_barrier()
    _stage(nsc2, m3, h2.at[c])
    plsc.subcore_barrier()

    for j in range(n_rounds):
        ci = j * NS + s

        @pl.when(ci < N_WB_CHUNKS)
        def _():
            r0 = ci * WB_CHUNK
            pltpu.sync_copy(acc.at[pl.ds(r0, WB_CHUNK)],
                            rows0.at[pl.ds(0, WB_CHUNK)])

            def relu_body(r, _):
                for q in range(H // 16):
                    sl = pl.ds(q * 16, 16)
                    rows0[r, sl] = jnp.maximum(rows0[r, sl], 0.0)
                return _
            lax.fori_loop(0, WB_CHUNK, relu_body, None)

            pltpu.sync_copy(rows0.at[pl.ds(0, WB_CHUNK)],
                            out_ref.at[c, pl.ds(r0, WB_CHUNK)])


def kernel(phi_indices, phi_values, phi_inverse_indices, phi_inverse_values,
           feature_indices, feature_values, weight_matrix,
           diagonal_weight_filter, dropout=0, device=0):
    f32 = jnp.float32

    e1 = feature_indices.shape[1]
    e2 = phi_indices.shape[1]
    grp = NS * K * SUP
    t1 = ((e1 + grp - 1) // grp) * grp
    t2 = ((e2 + grp - 1) // grp) * grp
    m1 = _pack_edges(feature_indices[1], feature_indices[0],
                     feature_values.astype(f32), t1, CH)
    m2 = _pack_edges(phi_inverse_indices[1], phi_inverse_indices[0],
                     phi_inverse_values.astype(f32), t2, N)
    m3 = _pack_edges(phi_indices[1], phi_indices[0],
                     phi_values.astype(f32), t2, N)

    w2 = weight_matrix.astype(f32).reshape(CH, 2, H).transpose(1, 0, 2)
    w2 = (w2.reshape(2, CH, 2, 2, 16).transpose(0, 1, 2, 4, 3)
          .reshape(2, CH, H).astype(jnp.bfloat16))
    diag = diagonal_weight_filter.astype(f32).reshape(N)

    mesh = plsc.VectorSubcoreMesh(core_axis_name="c", subcore_axis_name="s",
                                  num_cores=2, num_subcores=NS)
    out = pl.kernel(
        _body,
        out_type=(jax.ShapeDtypeStruct((2, N, H), f32),
                  jax.ShapeDtypeStruct((2, N, H), jnp.bfloat16),
                  jax.ShapeDtypeStruct((2, N, H), jnp.bfloat16)),
        mesh=mesh,
        compiler_params=pltpu.CompilerParams(needs_layout_passes=False,
                                             use_tc_tiling_on_sc=False),
        scratch_types=[
            pltpu.VMEM_SHARED((CH, H), jnp.bfloat16),
            pltpu.VMEM_SHARED((N, H), f32),
        ] + [pltpu.VMEM((K, H), jnp.bfloat16)] * NBUF
        + [pltpu.VMEM((K, H), f32)] * NBUF
        + [
            pltpu.VMEM((MB, 3, SUP, K), jnp.int32),
            pltpu.VMEM((WB_CHUNK,), f32),
            pltpu.VMEM((WB_CHUNK, H), f32),
            pltpu.VMEM((WB_CHUNK, H), jnp.bfloat16),
        ] + [pltpu.SemaphoreType.DMA] * (2 * NBUF + 1),
    )(m1, m2, m3, w2, diag)

    return out[0].transpose(1, 0, 2).reshape(N, CH)


# speedup vs baseline: 1.8749x; 1.8749x over previous
import jax
import jax.numpy as jnp
from jax import lax
from jax.experimental import pallas as pl
from jax.experimental.pallas import tpu as pltpu
from jax.experimental.pallas import tpu_sc as plsc

N = 10000
CH = 128
H = 64
NS = 16
K = 128
NBUF = 8
LOOK = 4
SUP = 8
MB = 3
WB_CHUNK = 80
N_WB_CHUNKS = N // WB_CHUNK


def _pack_edges(src, dst, val, total, src_mod):
    pad = total - src.shape[0]
    pad_idx = jnp.arange(pad, dtype=jnp.int32)
    src = jnp.concatenate([src.astype(jnp.int32), pad_idx % src_mod])
    dst = jnp.concatenate([dst.astype(jnp.int32), pad_idx % N])
    val = jnp.concatenate([val, jnp.zeros((pad,), val.dtype)])
    vbits = jax.lax.bitcast_convert_type(val.astype(jnp.float32), jnp.int32)
    nsc = total // (NS * K * SUP)
    packed = jnp.stack([x.reshape(NS * nsc, SUP, K) for x in (src, dst, vbits)],
                       axis=1)
    return packed


def _body(m1, m2, m3, w2, diag,
          out_ref, h1, h2,
          w_sp, acc,
          rows0, rows1, rows2, rows3, rows4, rows5, rows6, rows7,
          cbuf, dbuf, zbuf,
          gsem0, gsem1, gsem2, gsem3, gsem4, gsem5, gsem6, gsem7,
          ssem0, ssem1, ssem2, ssem3, ssem4, ssem5, ssem6, ssem7, msem):
    c = lax.axis_index("c")
    s = lax.axis_index("s")
    rows_list = (rows0, rows1, rows2, rows3, rows4, rows5, rows6, rows7)
    gsems = (gsem0, gsem1, gsem2, gsem3, gsem4, gsem5, gsem6, gsem7)
    ssems = (ssem0, ssem1, ssem2, ssem3, ssem4, ssem5, ssem6, ssem7)
    n_rounds = (N_WB_CHUNKS + NS - 1) // NS

    @pl.when(s == 0)
    def _():
        pltpu.sync_copy(w2.at[c], rows0)
        pltpu.sync_copy(rows0, w_sp)

    def zb(r, _):
        z = jnp.zeros((16,), jnp.float32)
        for q in range(H // 16):
            zbuf[r, pl.ds(q * 16, 16)] = z
        return _
    lax.fori_loop(0, WB_CHUNK, zb, None)

    for j in range(n_rounds):
        ci = j * NS + s

        @pl.when(ci < N_WB_CHUNKS)
        def _():
            pltpu.sync_copy(zbuf, acc.at[pl.ds(ci * WB_CHUNK, WB_CHUNK)])

    plsc.subcore_barrier()

    def _wait_rows(b, sems):
        pltpu.make_async_copy(w2.at[c], rows_list[b], sems[b]).wait()

    def _scale16(rows, r0, v16):
        for i in range(16):
            sv = v16[jnp.full((16,), i, jnp.int32)]
            for q in range(H // 16):
                sl = pl.ds(q * 16, 16)
                rows[r0 + i, sl] = rows[r0 + i, sl] * sv

    def _stage(nsc, meta, table_ref):
        nc = nsc * SUP
        base = s * nsc
        pltpu.sync_copy(meta.at[base], cbuf.at[0])
        if nsc > 1:
            pltpu.async_copy(meta.at[base + 1], cbuf.at[1], msem)

        for b in range(LOOK):
            pltpu.async_copy(table_ref.at[cbuf.at[0, 0, b]], rows_list[b],
                             gsems[b])

        def super_body(sb, _):
            b2 = lax.rem(sb, MB)
            b2n = lax.rem(sb + 1, MB)
            for j in range(SUP):
                k = sb * SUP + j
                b = j
                _wait_rows(b, gsems)
                bn = (j + LOOK) % NBUF

                @pl.when(k >= LOOK)
                def _():
                    _wait_rows(bn, ssems)

                if j == 3:
                    @pl.when(sb + 1 < nsc)
                    def _():
                        pltpu.make_async_copy(meta.at[base], cbuf.at[0],
                                              msem).wait()

                    @pl.when(sb + 2 < nsc)
                    def _():
                        pltpu.async_copy(meta.at[base + sb + 2],
                                         cbuf.at[lax.rem(sb + 2, MB)], msem)
                if j < SUP - LOOK:
                    pltpu.async_copy(table_ref.at[cbuf.at[b2, 0, j + LOOK]],
                                     rows_list[bn], gsems[bn])
                else:
                    @pl.when(sb + 1 < nsc)
                    def _():
                        pltpu.async_copy(
                            table_ref.at[cbuf.at[b2n, 0, j + LOOK - SUP]],
                            rows_list[bn], gsems[bn])
                def sc(jj, _):
                    for u in range(2):
                        v16i = cbuf[b2, 2, j, pl.ds(jj * 32 + u * 16, 16)]
                        v16 = plsc.bitcast(v16i, jnp.float32)
                        _scale16(rows_list[b], jj * 32 + u * 16, v16)
                    return _
                lax.fori_loop(0, K // 32, sc, None)
                pltpu.async_copy(rows_list[b], acc.at[cbuf.at[b2, 1, j]],
                                 ssems[b], add=True)
            return _
        lax.fori_loop(0, nsc, super_body, None)

        for k in range(nc - LOOK, nc):
            _wait_rows(k % NBUF, ssems)

    nsc1 = m1.shape[0] // NS
    nsc2 = m2.shape[0] // NS

    _stage(nsc1, m1, w_sp)
    plsc.subcore_barrier()
    for j in range(n_rounds):
        ci = j * NS + s

        @pl.when(ci < N_WB_CHUNKS)
        def _():
            r0 = ci * WB_CHUNK
            pltpu.sync_copy(acc.at[pl.ds(r0, WB_CHUNK)],
                            rows0.at[pl.ds(0, WB_CHUNK)])
            pltpu.sync_copy(rows0.at[pl.ds(0, WB_CHUNK)],
                            h1.at[c, pl.ds(r0, WB_CHUNK)])
            pltpu.sync_copy(zbuf, acc.at[pl.ds(r0, WB_CHUNK)])

    plsc.subcore_barrier()
    _stage(nsc2, m2, h1.at[c])
    plsc.subcore_barrier()
    for j in range(n_rounds):
        ci = j * NS + s

        @pl.when(ci < N_WB_CHUNKS)
        def _():
            r0 = ci * WB_CHUNK
            pltpu.sync_copy(acc.at[pl.ds(r0, WB_CHUNK)],
                            rows0.at[pl.ds(0, WB_CHUNK)])
            pltpu.sync_copy(diag.at[pl.ds(r0, WB_CHUNK)], dbuf)

            def dsc(jj, _):
                d16 = dbuf[pl.ds(jj * 16, 16)]
                _scale16(rows0, jj * 16, d16)
                return _
            lax.fori_loop(0, WB_CHUNK // 16, dsc, None)

            pltpu.sync_copy(rows0.at[pl.ds(0, WB_CHUNK)],
                            h2.at[c, pl.ds(r0, WB_CHUNK)])
            pltpu.sync_copy(zbuf, acc.at[pl.ds(r0, WB_CHUNK)])

    plsc.subcore_barrier()
    _stage(nsc2, m3, h2.at[c])
    plsc.subcore_barrier()

    for j in range(n_rounds):
        ci = j * NS + s

        @pl.when(ci < N_WB_CHUNKS)
        def _():
            r0 = ci * WB_CHUNK
            pltpu.sync_copy(acc.at[pl.ds(r0, WB_CHUNK)],
                            rows0.at[pl.ds(0, WB_CHUNK)])

            def relu_body(r, _):
                for q in range(H // 16):
                    sl = pl.ds(q * 16, 16)
                    rows0[r, sl] = jnp.maximum(rows0[r, sl], 0.0)
                return _
            lax.fori_loop(0, WB_CHUNK, relu_body, None)

            pltpu.sync_copy(rows0.at[pl.ds(0, WB_CHUNK)],
                            out_ref.at[c, pl.ds(r0, WB_CHUNK)])


def kernel(phi_indices, phi_values, phi_inverse_indices, phi_inverse_values,
           feature_indices, feature_values, weight_matrix,
           diagonal_weight_filter, dropout=0, device=0):
    f32 = jnp.float32

    e1 = feature_indices.shape[1]
    e2 = phi_indices.shape[1]
    grp = NS * K * SUP
    t1 = ((e1 + grp - 1) // grp) * grp
    t2 = ((e2 + grp - 1) // grp) * grp
    m1 = _pack_edges(feature_indices[1], feature_indices[0],
                     feature_values.astype(f32), t1, CH)
    m2 = _pack_edges(phi_inverse_indices[1], phi_inverse_indices[0],
                     phi_inverse_values.astype(f32), t2, N)
    m3 = _pack_edges(phi_indices[1], phi_indices[0],
                     phi_values.astype(f32), t2, N)

    w2 = weight_matrix.astype(f32).reshape(CH, 2, H).transpose(1, 0, 2)
    diag = diagonal_weight_filter.astype(f32).reshape(N)

    mesh = plsc.VectorSubcoreMesh(core_axis_name="c", subcore_axis_name="s",
                                  num_cores=2, num_subcores=NS)
    out = pl.kernel(
        _body,
        out_type=(jax.ShapeDtypeStruct((2, N, H), f32),
                  jax.ShapeDtypeStruct((2, N, H), f32),
                  jax.ShapeDtypeStruct((2, N, H), f32)),
        mesh=mesh,
        compiler_params=pltpu.CompilerParams(needs_layout_passes=False,
                                             use_tc_tiling_on_sc=False),
        scratch_types=[
            pltpu.VMEM_SHARED((CH, H), f32),
            pltpu.VMEM_SHARED((N, H), f32),
        ] + [pltpu.VMEM((K, H), f32)] * NBUF
        + [
            pltpu.VMEM((MB, 3, SUP, K), jnp.int32),
            pltpu.VMEM((WB_CHUNK,), f32),
            pltpu.VMEM((WB_CHUNK, H), f32),
        ] + [pltpu.SemaphoreType.DMA] * (2 * NBUF + 1),
    )(m1, m2, m3, w2, diag)

    return out[0].transpose(1, 0, 2).reshape(N, CH)
